# Initial kernel scaffold; baseline (speedup 1.0000x reference)
import jax
import jax.numpy as jnp
from jax.experimental import pallas as pl


def kernel(pred_probs, target_mask):
    raise NotImplementedError("write your pallas kernel here")


# speedup vs baseline: 121.7405x; 121.7405x over previous
import jax
import jax.numpy as jnp
from jax import lax
from jax.experimental import pallas as pl
from jax.experimental.pallas import tpu as pltpu
from jax.experimental.pallas import tpu_sc as plsc

THETA = 180
LANES = 16
NGROUPS = 12
H = W = 384
NPIX = H * W
BATCH = 2
N_RHO = 1088
RHO_OFF = float(N_RHO // 2)
ACTIVE = BATCH * NGROUPS
CHUNK_ROWS = 32
CHUNK = CHUNK_ROWS * W
NCHUNKS = H // CHUNK_ROWS
XU = 4
TOTAL_BINS = float(BATCH * THETA * N_RHO)


def _sc_body(pred_hbm, tgt_hbm, cos_hbm, sin_hbm, out_hbm,
             pred_buf, tgt_buf, d_buf, cbuf, sbuf, xc_tab, pbuf, acc):
    wid = lax.axis_index("s") * 2 + lax.axis_index("c")
    lane = lax.broadcasted_iota(jnp.int32, (LANES,), 0)
    lanebase = lane * N_RHO
    zero_v = jnp.zeros((LANES,), jnp.float32)

    pbuf[...] = zero_v

    @pl.when(wid < ACTIVE)
    def _():
        b = wid // NGROUPS
        g = wid % NGROUPS
        pltpu.sync_copy(cos_hbm.at[pl.ds(g * LANES, LANES)], cbuf)
        pltpu.sync_copy(sin_hbm.at[pl.ds(g * LANES, LANES)], sbuf)
        c_v = cbuf[...]
        s_v = sbuf[...]

        def tbody(x, xf):
            xc_tab[pl.ds(x * LANES, LANES)] = c_v * xf
            return xf + 1.0
        lax.fori_loop(0, W, tbody, jnp.float32(0.0))

        def zbody(j, _):
            acc[pl.ds(j * LANES, LANES)] = zero_v
            return 0
        lax.fori_loop(0, LANES * N_RHO // LANES, zbody, 0)

        def chunk_body(ck, yf0):
            off = b * NPIX + ck * CHUNK
            pltpu.sync_copy(pred_hbm.at[pl.ds(off, CHUNK)], pred_buf)
            pltpu.sync_copy(tgt_hbm.at[pl.ds(off, CHUNK)], tgt_buf)

            def dbody(i, _):
                pv = pred_buf[pl.ds(i * LANES, LANES)]
                tv = tgt_buf[pl.ds(i * LANES, LANES)]
                d_buf[pl.ds(i * LANES, LANES)] = (
                    jnp.where(pv > 0.5, 1.0, 0.0)
                    - jnp.where(tv > 0.5, 1.0, 0.0))
                return 0
            lax.fori_loop(0, CHUNK // LANES, dbody, 0)

            def row_body(r, yf):
                sy_v = s_v * yf

                @plsc.parallel_loop(0, W // LANES, unroll=2)
                def x_body(i):
                    dv = d_buf[pl.ds(r * W + i * LANES, LANES)]
                    for k in range(LANES):
                        xc = xc_tab[pl.ds(i * (LANES * LANES) + k * LANES,
                                          LANES)]
                        rho = (xc + sy_v) + RHO_OFF
                        plsc.addupdate_scatter(
                            acc, [lanebase + rho.astype(jnp.int32)],
                            jnp.full((LANES,), dv[k]))
                return yf + 1.0
            yf1 = lax.fori_loop(0, CHUNK_ROWS, row_body, yf0)
            return yf1
        lax.fori_loop(0, NCHUNKS, chunk_body, jnp.float32(0.0))

        def lrow(r, _):
            def lcol(cc, pv):
                return pv + jnp.abs(acc[pl.ds(r * N_RHO + cc * LANES, LANES)])
            prow = lax.fori_loop(0, N_RHO // LANES, lcol, zero_v)
            scale = jnp.where(g * LANES + r < THETA, 1.0, 0.0)
            pbuf[...] = pbuf[...] + prow * scale
            return 0
        lax.fori_loop(0, LANES, lrow, 0)

    pltpu.sync_copy(pbuf, out_hbm.at[pl.ds(wid * LANES, LANES)])


def kernel(pred_probs, target_mask):
    thetas = jnp.linspace(0.0, jnp.pi, THETA, dtype=jnp.float32)
    pad = jnp.zeros((NGROUPS * LANES - THETA,), jnp.float32)
    cos_p = jnp.concatenate([jnp.cos(thetas), pad])
    sin_p = jnp.concatenate([jnp.sin(thetas), pad])
    pred = pred_probs.reshape(-1)
    tgt = target_mask.reshape(-1)

    mesh = plsc.VectorSubcoreMesh(core_axis_name="c", subcore_axis_name="s")
    partials = pl.kernel(
        _sc_body,
        out_type=jax.ShapeDtypeStruct((32 * LANES,), jnp.float32),
        mesh=mesh,
        compiler_params=pltpu.CompilerParams(needs_layout_passes=False),
        scratch_types=[
            pltpu.VMEM((CHUNK,), jnp.float32),
            pltpu.VMEM((CHUNK,), jnp.float32),
            pltpu.VMEM((CHUNK,), jnp.float32),
            pltpu.VMEM((LANES,), jnp.float32),
            pltpu.VMEM((LANES,), jnp.float32),
            pltpu.VMEM((W * LANES,), jnp.float32),
            pltpu.VMEM((LANES,), jnp.float32),
            pltpu.VMEM((LANES * N_RHO,), jnp.float32),
        ],
    )(pred, tgt, cos_p, sin_p)
    return partials.sum() / TOTAL_BINS
